# speedup vs baseline: 2.4270x; 2.4270x over previous
import functools

import jax
import jax.numpy as jnp
from jax import lax
from jax.experimental import pallas as pl
from jax.experimental.pallas import tpu as pltpu
from jax.experimental.pallas import tpu_sc as plsc

VOCAB = 100000
HIDDEN = 128
N_IDS = 4096 * 50

_ROW_BLOCK = 2000


def _linear_body(e_ref, w_ref, b_ref, o_ref):
    acc = lax.dot_general(
        e_ref[...], w_ref[...],
        dimension_numbers=(((1,), (1,)), ((), ())),
        preferred_element_type=jnp.float32,
    )
    o_ref[...] = acc + b_ref[...]


def _transform_table(embed_table, W, b):
    grid = VOCAB // _ROW_BLOCK
    return pl.pallas_call(
        _linear_body,
        grid=(grid,),
        in_specs=[
            pl.BlockSpec((_ROW_BLOCK, HIDDEN), lambda i: (i, 0)),
            pl.BlockSpec((HIDDEN, HIDDEN), lambda i: (0, 0)),
            pl.BlockSpec((1, HIDDEN), lambda i: (0, 0)),
        ],
        out_specs=pl.BlockSpec((_ROW_BLOCK, HIDDEN), lambda i: (i, 0)),
        out_shape=jax.ShapeDtypeStruct((VOCAB, HIDDEN), jnp.float32),
    )(embed_table, W, b.reshape(1, HIDDEN))


def _make_gather():
    info = plsc.get_sparse_core_info()
    nc, ns = info.num_cores, info.num_subcores
    nw = nc * ns
    b_per_w = N_IDS // nw
    chunk = 320
    n_chunks = b_per_w // chunk
    mesh = plsc.VectorSubcoreMesh(core_axis_name="c", subcore_axis_name="s")

    @functools.partial(
        pl.kernel,
        mesh=mesh,
        out_type=jax.ShapeDtypeStruct((N_IDS, HIDDEN), jnp.float32),
        scratch_types=[
            pltpu.VMEM((b_per_w,), jnp.int32),
            pltpu.VMEM((chunk, HIDDEN), jnp.float32),
            pltpu.SemaphoreType.DMA,
        ],
    )
    def gather(table_hbm, idx_hbm, out_hbm, idx_v, rows_v, sem):
        wid = lax.axis_index("s") * nc + lax.axis_index("c")
        base = wid * b_per_w
        pltpu.sync_copy(idx_hbm.at[pl.ds(base, b_per_w)], idx_v)

        def body(i, carry):
            off = i * chunk
            pltpu.async_copy(
                table_hbm.at[idx_v.at[pl.ds(off, chunk)]], rows_v, sem
            ).wait()
            pltpu.sync_copy(rows_v, out_hbm.at[pl.ds(base + off, chunk)])
            return carry

        lax.fori_loop(0, n_chunks, body, 0)

    return gather


_gather = _make_gather()


def kernel(input_ids, embed_table, W, b):
    eprime = _transform_table(embed_table, W, b)
    flat_ids = input_ids.reshape(-1).astype(jnp.int32)
    out_flat = _gather(eprime, flat_ids)
    return out_flat.reshape(input_ids.shape[0], input_ids.shape[1], HIDDEN)


# speedup vs baseline: 2.5093x; 1.0339x over previous
import functools

import jax
import jax.numpy as jnp
from jax import lax
from jax.experimental import pallas as pl
from jax.experimental.pallas import tpu as pltpu
from jax.experimental.pallas import tpu_sc as plsc

VOCAB = 100000
HIDDEN = 128
N_IDS = 4096 * 50

_ROW_BLOCK = 2000


def _linear_body(e_ref, w_ref, b_ref, o_ref):
    acc = lax.dot_general(
        e_ref[...], w_ref[...],
        dimension_numbers=(((1,), (1,)), ((), ())),
        preferred_element_type=jnp.float32,
    )
    o_ref[...] = acc + b_ref[...]


def _transform_table(embed_table, W, b):
    grid = VOCAB // _ROW_BLOCK
    return pl.pallas_call(
        _linear_body,
        grid=(grid,),
        in_specs=[
            pl.BlockSpec((_ROW_BLOCK, HIDDEN), lambda i: (i, 0)),
            pl.BlockSpec((HIDDEN, HIDDEN), lambda i: (0, 0)),
            pl.BlockSpec((1, HIDDEN), lambda i: (0, 0)),
        ],
        out_specs=pl.BlockSpec((_ROW_BLOCK, HIDDEN), lambda i: (i, 0)),
        out_shape=jax.ShapeDtypeStruct((VOCAB, HIDDEN), jnp.float32),
    )(embed_table, W, b.reshape(1, HIDDEN))


def _make_gather():
    info = plsc.get_sparse_core_info()
    nc, ns = info.num_cores, info.num_subcores
    nw = nc * ns
    b_per_w = N_IDS // nw
    chunk = 160
    nbuf = 4
    n_chunks = b_per_w // chunk
    n_groups = n_chunks // nbuf
    mesh = plsc.VectorSubcoreMesh(core_axis_name="c", subcore_axis_name="s")

    scratch = [pltpu.VMEM((b_per_w,), jnp.int32)]
    scratch += [pltpu.VMEM((chunk, HIDDEN), jnp.float32) for _ in range(nbuf)]
    scratch += [pltpu.SemaphoreType.DMA for _ in range(2 * nbuf)]

    @functools.partial(
        pl.kernel,
        mesh=mesh,
        out_type=jax.ShapeDtypeStruct((N_IDS, HIDDEN), jnp.float32),
        scratch_types=scratch,
    )
    def gather(table_hbm, idx_hbm, out_hbm, idx_v, *bufs_and_sems):
        bufs = bufs_and_sems[:nbuf]
        gsems = bufs_and_sems[nbuf:2 * nbuf]
        ssems = bufs_and_sems[2 * nbuf:]
        wid = lax.axis_index("s") * nc + lax.axis_index("c")
        base = wid * b_per_w
        pltpu.sync_copy(idx_hbm.at[pl.ds(base, b_per_w)], idx_v)

        def g_copy(i, k):
            return pltpu.make_async_copy(
                table_hbm.at[idx_v.at[pl.ds(i * chunk, chunk)]],
                bufs[k], gsems[k])

        def s_copy(i, k):
            return pltpu.make_async_copy(
                bufs[k], out_hbm.at[pl.ds(base + i * chunk, chunk)],
                ssems[k])

        for k in range(nbuf):
            g_copy(k, k).start()

        def outer(j, carry):
            for k in range(nbuf):
                i = j * nbuf + k
                g_copy(i, k).wait()
                s_copy(i, k).start()
                s_copy(i, k).wait()

                @pl.when(j < n_groups - 1)
                def _():
                    g_copy(i + nbuf, k).start()
            return carry

        lax.fori_loop(0, n_groups, outer, 0)

    return gather


_gather = _make_gather()


def kernel(input_ids, embed_table, W, b):
    eprime = _transform_table(embed_table, W, b)
    flat_ids = input_ids.reshape(-1).astype(jnp.int32)
    out_flat = _gather(eprime, flat_ids)
    return out_flat.reshape(input_ids.shape[0], input_ids.shape[1], HIDDEN)


# speedup vs baseline: 2.5169x; 1.0030x over previous
import functools

import jax
import jax.numpy as jnp
from jax import lax
from jax.experimental import pallas as pl
from jax.experimental.pallas import tpu as pltpu
from jax.experimental.pallas import tpu_sc as plsc

VOCAB = 100000
HIDDEN = 128
N_IDS = 4096 * 50

_ROW_BLOCK = 2000


def _linear_body(e_ref, w_ref, b_ref, o_ref):
    acc = lax.dot_general(
        e_ref[...], w_ref[...],
        dimension_numbers=(((1,), (1,)), ((), ())),
        preferred_element_type=jnp.float32,
    )
    o_ref[...] = acc + b_ref[...]


def _transform_table(embed_table, W, b):
    grid = VOCAB // _ROW_BLOCK
    return pl.pallas_call(
        _linear_body,
        grid=(grid,),
        in_specs=[
            pl.BlockSpec((_ROW_BLOCK, HIDDEN), lambda i: (i, 0)),
            pl.BlockSpec((HIDDEN, HIDDEN), lambda i: (0, 0)),
            pl.BlockSpec((1, HIDDEN), lambda i: (0, 0)),
        ],
        out_specs=pl.BlockSpec((_ROW_BLOCK, HIDDEN), lambda i: (i, 0)),
        out_shape=jax.ShapeDtypeStruct((VOCAB, HIDDEN), jnp.float32),
    )(embed_table, W, b.reshape(1, HIDDEN))


def _make_gather():
    info = plsc.get_sparse_core_info()
    nc, ns = info.num_cores, info.num_subcores
    nw = nc * ns
    b_per_w = N_IDS // nw
    chunk = 160
    nbuf = 4
    n_chunks = b_per_w // chunk
    n_groups = n_chunks // nbuf
    mesh = plsc.VectorSubcoreMesh(core_axis_name="c", subcore_axis_name="s")

    scratch = [pltpu.VMEM((b_per_w,), jnp.int32)]
    scratch += [pltpu.VMEM((chunk, HIDDEN), jnp.float32) for _ in range(nbuf)]
    scratch += [pltpu.SemaphoreType.DMA for _ in range(2 * nbuf)]

    @functools.partial(
        pl.kernel,
        mesh=mesh,
        out_type=jax.ShapeDtypeStruct((N_IDS, HIDDEN), jnp.float32),
        scratch_types=scratch,
        compiler_params=pltpu.CompilerParams(use_tc_tiling_on_sc=True),
    )
    def gather(table_hbm, idx_hbm, out_hbm, idx_v, *bufs_and_sems):
        bufs = bufs_and_sems[:nbuf]
        gsems = bufs_and_sems[nbuf:2 * nbuf]
        ssems = bufs_and_sems[2 * nbuf:]
        wid = lax.axis_index("s") * nc + lax.axis_index("c")
        base = wid * b_per_w
        pltpu.sync_copy(idx_hbm.at[pl.ds(base, b_per_w)], idx_v)

        def g_copy(i, k):
            return pltpu.make_async_copy(
                table_hbm.at[idx_v.at[pl.ds(i * chunk, chunk)]],
                bufs[k], gsems[k])

        def s_copy(i, k):
            return pltpu.make_async_copy(
                bufs[k], out_hbm.at[pl.ds(base + i * chunk, chunk)],
                ssems[k])

        for k in range(nbuf):
            g_copy(k, k).start()

        def outer(j, carry):
            for k in range(nbuf):
                i = j * nbuf + k
                g_copy(i, k).wait()
                s_copy(i, k).start()
                s_copy(i, k).wait()

                @pl.when(j < n_groups - 1)
                def _():
                    g_copy(i + nbuf, k).start()
            return carry

        lax.fori_loop(0, n_groups, outer, 0)

    return gather


_gather = _make_gather()


def kernel(input_ids, embed_table, W, b):
    eprime = _transform_table(embed_table, W, b)
    flat_ids = input_ids.reshape(-1).astype(jnp.int32)
    out_flat = _gather(eprime, flat_ids)
    return out_flat.reshape(input_ids.shape[0], input_ids.shape[1], HIDDEN)


# speedup vs baseline: 3.3139x; 1.3167x over previous
import functools

import jax
import jax.numpy as jnp
from jax import lax
from jax.experimental import pallas as pl
from jax.experimental.pallas import tpu as pltpu
from jax.experimental.pallas import tpu_sc as plsc

VOCAB = 100000
HIDDEN = 128
BATCH = 4096
SEQ = 50
N_IDS = BATCH * SEQ

_B_BLK = 64


def _linear_body(g_ref, w_ref, b_ref, o_ref):
    acc = lax.dot_general(
        g_ref[...], w_ref[...],
        dimension_numbers=(((1,), (1,)), ((), ())),
        preferred_element_type=jnp.float32,
    )
    o_ref[...] = (acc + b_ref[...]).reshape(_B_BLK, SEQ, HIDDEN)


def _linear(gathered, W, b):
    grid = BATCH // _B_BLK
    return pl.pallas_call(
        _linear_body,
        grid=(grid,),
        in_specs=[
            pl.BlockSpec((_B_BLK * SEQ, HIDDEN), lambda i: (i, 0)),
            pl.BlockSpec((HIDDEN, HIDDEN), lambda i: (0, 0)),
            pl.BlockSpec((1, HIDDEN), lambda i: (0, 0)),
        ],
        out_specs=pl.BlockSpec((_B_BLK, SEQ, HIDDEN), lambda i: (i, 0, 0)),
        out_shape=jax.ShapeDtypeStruct((BATCH, SEQ, HIDDEN), jnp.float32),
    )(gathered, W, b.reshape(1, HIDDEN))


def _make_gather():
    info = plsc.get_sparse_core_info()
    nc, ns = info.num_cores, info.num_subcores
    nw = nc * ns
    b_per_w = N_IDS // nw
    chunk = 160
    nbuf = 4
    n_chunks = b_per_w // chunk
    n_groups = n_chunks // nbuf
    mesh = plsc.VectorSubcoreMesh(core_axis_name="c", subcore_axis_name="s")

    scratch = [pltpu.VMEM((b_per_w,), jnp.int32)]
    scratch += [pltpu.VMEM((chunk, HIDDEN), jnp.float32) for _ in range(nbuf)]
    scratch += [pltpu.SemaphoreType.DMA for _ in range(2 * nbuf)]

    @functools.partial(
        pl.kernel,
        mesh=mesh,
        out_type=jax.ShapeDtypeStruct((N_IDS, HIDDEN), jnp.float32),
        scratch_types=scratch,
    )
    def gather(table_hbm, idx_hbm, out_hbm, idx_v, *bufs_and_sems):
        bufs = bufs_and_sems[:nbuf]
        gsems = bufs_and_sems[nbuf:2 * nbuf]
        ssems = bufs_and_sems[2 * nbuf:]
        wid = lax.axis_index("s") * nc + lax.axis_index("c")
        base = wid * b_per_w
        pltpu.sync_copy(idx_hbm.at[pl.ds(base, b_per_w)], idx_v)

        def g_copy(i, k):
            return pltpu.make_async_copy(
                table_hbm.at[idx_v.at[pl.ds(i * chunk, chunk)]],
                bufs[k], gsems[k])

        def s_copy(i, k):
            return pltpu.make_async_copy(
                bufs[k], out_hbm.at[pl.ds(base + i * chunk, chunk)],
                ssems[k])

        for k in range(nbuf):
            g_copy(k, k).start()

        def outer(j, carry):
            for k in range(nbuf):
                i = j * nbuf + k
                g_copy(i, k).wait()
                s_copy(i, k).start()
                s_copy(i, k).wait()

                @pl.when(j < n_groups - 1)
                def _():
                    g_copy(i + nbuf, k).start()
            return carry

        lax.fori_loop(0, n_groups, outer, 0)

    return gather


_gather = _make_gather()


def kernel(input_ids, embed_table, W, b):
    flat_ids = input_ids.reshape(-1).astype(jnp.int32)
    gathered = _gather(embed_table, flat_ids)
    return _linear(gathered, W, b)


# speedup vs baseline: 4.8831x; 1.4735x over previous
import functools

import jax
import jax.numpy as jnp
from jax import lax
from jax.experimental import pallas as pl
from jax.experimental.pallas import tpu as pltpu
from jax.experimental.pallas import tpu_sc as plsc

VOCAB = 100000
HIDDEN = 128
BATCH = 4096
SEQ = 50
N_IDS = BATCH * SEQ

_B_BLK = 64


def _linear_body(g_ref, w_ref, b_ref, o_ref):
    x = g_ref[...].reshape(SEQ * _B_BLK, HIDDEN)
    acc = lax.dot_general(
        x, w_ref[...],
        dimension_numbers=(((1,), (1,)), ((), ())),
        preferred_element_type=jnp.float32,
    )
    o_ref[...] = (acc + b_ref[...]).reshape(SEQ, _B_BLK, HIDDEN)


def _linear(gathered_sm, W, b):
    grid = BATCH // _B_BLK
    return pl.pallas_call(
        _linear_body,
        grid=(grid,),
        in_specs=[
            pl.BlockSpec((SEQ, _B_BLK, HIDDEN), lambda i: (0, i, 0)),
            pl.BlockSpec((HIDDEN, HIDDEN), lambda i: (0, 0)),
            pl.BlockSpec((1, HIDDEN), lambda i: (0, 0)),
        ],
        out_specs=pl.BlockSpec((SEQ, _B_BLK, HIDDEN), lambda i: (0, i, 0)),
        out_shape=jax.ShapeDtypeStruct((SEQ, BATCH, HIDDEN), jnp.float32),
    )(gathered_sm, W, b.reshape(1, HIDDEN))


def _make_gather():
    info = plsc.get_sparse_core_info()
    nc, ns = info.num_cores, info.num_subcores
    nw = nc * ns
    b_per_w = N_IDS // nw
    chunk = 160
    nbuf = 4
    n_chunks = b_per_w // chunk
    n_groups = n_chunks // nbuf
    mesh = plsc.VectorSubcoreMesh(core_axis_name="c", subcore_axis_name="s")

    scratch = [pltpu.VMEM((b_per_w,), jnp.int32)]
    scratch += [pltpu.VMEM((chunk, HIDDEN), jnp.float32) for _ in range(nbuf)]
    scratch += [pltpu.SemaphoreType.DMA for _ in range(2 * nbuf)]

    @functools.partial(
        pl.kernel,
        mesh=mesh,
        out_type=jax.ShapeDtypeStruct((N_IDS, HIDDEN), jnp.float32),
        scratch_types=scratch,
    )
    def gather(table_hbm, idx_hbm, out_hbm, idx_v, *bufs_and_sems):
        bufs = bufs_and_sems[:nbuf]
        gsems = bufs_and_sems[nbuf:2 * nbuf]
        ssems = bufs_and_sems[2 * nbuf:]
        wid = lax.axis_index("s") * nc + lax.axis_index("c")
        base = wid * b_per_w
        pltpu.sync_copy(idx_hbm.at[pl.ds(base, b_per_w)], idx_v)

        def g_copy(i, k):
            return pltpu.make_async_copy(
                table_hbm.at[idx_v.at[pl.ds(i * chunk, chunk)]],
                bufs[k], gsems[k])

        def s_copy(i, k):
            return pltpu.make_async_copy(
                bufs[k], out_hbm.at[pl.ds(base + i * chunk, chunk)],
                ssems[k])

        for k in range(nbuf):
            g_copy(k, k).start()

        def outer(j, carry):
            for k in range(nbuf):
                i = j * nbuf + k
                g_copy(i, k).wait()
                s_copy(i, k).start()
                s_copy(i, k).wait()

                @pl.when(j < n_groups - 1)
                def _():
                    g_copy(i + nbuf, k).start()
            return carry

        lax.fori_loop(0, n_groups, outer, 0)

    return gather


_gather = _make_gather()


def kernel(input_ids, embed_table, W, b):
    flat_ids = input_ids.T.reshape(-1).astype(jnp.int32)
    gathered = _gather(embed_table, flat_ids)
    out_sm = _linear(gathered.reshape(SEQ, BATCH, HIDDEN), W, b)
    return out_sm.transpose(1, 0, 2)


# speedup vs baseline: 5.8837x; 1.2049x over previous
import functools

import jax
import jax.numpy as jnp
from jax import lax
from jax.experimental import pallas as pl
from jax.experimental.pallas import tpu as pltpu
from jax.experimental.pallas import tpu_sc as plsc

VOCAB = 100000
HIDDEN = 128
BATCH = 4096
SEQ = 50
N_IDS = BATCH * SEQ

_ROW_BLOCK = 2000


def _linear_body(e_ref, w_ref, b_ref, o_ref):
    acc = lax.dot_general(
        e_ref[...], w_ref[...],
        dimension_numbers=(((1,), (1,)), ((), ())),
        preferred_element_type=jnp.float32,
    )
    o_ref[...] = acc + b_ref[...]


def _transform_table(embed_table, W, b):
    grid = VOCAB // _ROW_BLOCK
    return pl.pallas_call(
        _linear_body,
        grid=(grid,),
        in_specs=[
            pl.BlockSpec((_ROW_BLOCK, HIDDEN), lambda i: (i, 0)),
            pl.BlockSpec((HIDDEN, HIDDEN), lambda i: (0, 0)),
            pl.BlockSpec((1, HIDDEN), lambda i: (0, 0)),
        ],
        out_specs=pl.BlockSpec((_ROW_BLOCK, HIDDEN), lambda i: (i, 0)),
        out_shape=jax.ShapeDtypeStruct((VOCAB, HIDDEN), jnp.float32),
    )(embed_table, W, b.reshape(1, HIDDEN))


def _make_gather():
    info = plsc.get_sparse_core_info()
    nc, ns = info.num_cores, info.num_subcores
    nw = nc * ns
    b_per_w = N_IDS // nw
    chunk = 160
    nbuf = 4
    n_chunks = b_per_w // chunk
    n_groups = n_chunks // nbuf
    mesh = plsc.VectorSubcoreMesh(core_axis_name="c", subcore_axis_name="s")

    scratch = [pltpu.VMEM((b_per_w,), jnp.int32)]
    scratch += [pltpu.VMEM((chunk, HIDDEN), jnp.float32) for _ in range(nbuf)]
    scratch += [pltpu.SemaphoreType.DMA for _ in range(2 * nbuf)]

    @functools.partial(
        pl.kernel,
        mesh=mesh,
        out_type=jax.ShapeDtypeStruct((N_IDS, HIDDEN), jnp.float32),
        scratch_types=scratch,
    )
    def gather(table_hbm, idx_hbm, out_hbm, idx_v, *bufs_and_sems):
        bufs = bufs_and_sems[:nbuf]
        gsems = bufs_and_sems[nbuf:2 * nbuf]
        ssems = bufs_and_sems[2 * nbuf:]
        wid = lax.axis_index("s") * nc + lax.axis_index("c")
        base = wid * b_per_w
        pltpu.sync_copy(idx_hbm.at[pl.ds(base, b_per_w)], idx_v)

        def g_copy(i, k):
            return pltpu.make_async_copy(
                table_hbm.at[idx_v.at[pl.ds(i * chunk, chunk)]],
                bufs[k], gsems[k])

        def s_copy(i, k):
            return pltpu.make_async_copy(
                bufs[k], out_hbm.at[pl.ds(base + i * chunk, chunk)],
                ssems[k])

        for k in range(nbuf):
            g_copy(k, k).start()

        def outer(j, carry):
            for k in range(nbuf):
                i = j * nbuf + k
                g_copy(i, k).wait()
                s_copy(i, k).start()
                s_copy(i, k).wait()

                @pl.when(j < n_groups - 1)
                def _():
                    g_copy(i + nbuf, k).start()
            return carry

        lax.fori_loop(0, n_groups, outer, 0)

    return gather


_gather = _make_gather()


def kernel(input_ids, embed_table, W, b):
    eprime = _transform_table(embed_table, W, b)
    flat_ids = input_ids.T.reshape(-1).astype(jnp.int32)
    out_flat = _gather(eprime, flat_ids)
    return out_flat.reshape(SEQ, BATCH, HIDDEN).transpose(1, 0, 2)


# speedup vs baseline: 6.6271x; 1.1263x over previous
import functools

import jax
import jax.numpy as jnp
from jax import lax
from jax.experimental import pallas as pl
from jax.experimental.pallas import tpu as pltpu
from jax.experimental.pallas import tpu_sc as plsc

VOCAB = 100000
HIDDEN = 128
BATCH = 4096
SEQ = 50
N_IDS = BATCH * SEQ

_ROW_BLOCK = 5000


def _linear_body(e_ref, w_ref, b_ref, o_ref):
    acc = lax.dot_general(
        e_ref[...], w_ref[...],
        dimension_numbers=(((1,), (1,)), ((), ())),
        preferred_element_type=jnp.float32,
    )
    o_ref[...] = acc + b_ref[...]


def _transform_table(embed_table, W, b):
    grid = VOCAB // _ROW_BLOCK
    return pl.pallas_call(
        _linear_body,
        grid=(grid,),
        in_specs=[
            pl.BlockSpec((_ROW_BLOCK, HIDDEN), lambda i: (i, 0)),
            pl.BlockSpec((HIDDEN, HIDDEN), lambda i: (0, 0)),
            pl.BlockSpec((1, HIDDEN), lambda i: (0, 0)),
        ],
        out_specs=pl.BlockSpec((_ROW_BLOCK, HIDDEN), lambda i: (i, 0)),
        out_shape=jax.ShapeDtypeStruct((VOCAB, HIDDEN), jnp.float32),
    )(embed_table, W, b.reshape(1, HIDDEN))


def _make_gather():
    info = plsc.get_sparse_core_info()
    nc, ns = info.num_cores, info.num_subcores
    nw = nc * ns
    b_per_w = N_IDS // nw
    chunk = 160
    nbuf = 4
    n_chunks = b_per_w // chunk
    n_groups = n_chunks // nbuf
    mesh = plsc.VectorSubcoreMesh(core_axis_name="c", subcore_axis_name="s")

    scratch = [pltpu.VMEM((b_per_w,), jnp.int32)]
    scratch += [pltpu.VMEM((chunk, HIDDEN), jnp.float32) for _ in range(nbuf)]
    scratch += [pltpu.SemaphoreType.DMA for _ in range(2 * nbuf)]

    @functools.partial(
        pl.kernel,
        mesh=mesh,
        out_type=jax.ShapeDtypeStruct((N_IDS, HIDDEN), jnp.float32),
        scratch_types=scratch,
    )
    def gather(table_hbm, idx_hbm, out_hbm, idx_v, *bufs_and_sems):
        bufs = bufs_and_sems[:nbuf]
        gsems = bufs_and_sems[nbuf:2 * nbuf]
        ssems = bufs_and_sems[2 * nbuf:]
        wid = lax.axis_index("s") * nc + lax.axis_index("c")
        base = wid * b_per_w
        pltpu.sync_copy(idx_hbm.at[pl.ds(base, b_per_w)], idx_v)

        def g_copy(i, k):
            return pltpu.make_async_copy(
                table_hbm.at[idx_v.at[pl.ds(i * chunk, chunk)]],
                bufs[k], gsems[k])

        def s_copy(i, k):
            return pltpu.make_async_copy(
                bufs[k], out_hbm.at[pl.ds(base + i * chunk, chunk)],
                ssems[k])

        for k in range(nbuf):
            g_copy(k, k).start()

        def outer(j, carry):
            for k in range(nbuf):
                i = j * nbuf + k
                g_copy(i, k).wait()
                s_copy(i, k).start()
                s_copy(i, k).wait()

                @pl.when(j < n_groups - 1)
                def _():
                    g_copy(i + nbuf, k).start()
            return carry

        lax.fori_loop(0, n_groups, outer, 0)

    return gather


_gather = _make_gather()


def kernel(input_ids, embed_table, W, b):
    eprime = _transform_table(embed_table, W, b)
    flat_ids = input_ids.T.reshape(-1).astype(jnp.int32)
    out_flat = _gather(eprime, flat_ids)
    return out_flat.reshape(SEQ, BATCH, HIDDEN).transpose(1, 0, 2)


# speedup vs baseline: 6.8870x; 1.0392x over previous
import functools

import jax
import jax.numpy as jnp
from jax import lax
from jax.experimental import pallas as pl
from jax.experimental.pallas import tpu as pltpu
from jax.experimental.pallas import tpu_sc as plsc

VOCAB = 100000
HIDDEN = 128
BATCH = 4096
SEQ = 50
N_IDS = BATCH * SEQ

_ROW_BLOCK = 10000


def _linear_body(e_ref, w_ref, b_ref, o_ref):
    acc = lax.dot_general(
        e_ref[...], w_ref[...],
        dimension_numbers=(((1,), (1,)), ((), ())),
        preferred_element_type=jnp.float32,
    )
    o_ref[...] = acc + b_ref[...]


def _transform_table(embed_table, W, b):
    grid = VOCAB // _ROW_BLOCK
    return pl.pallas_call(
        _linear_body,
        grid=(grid,),
        in_specs=[
            pl.BlockSpec((_ROW_BLOCK, HIDDEN), lambda i: (i, 0)),
            pl.BlockSpec((HIDDEN, HIDDEN), lambda i: (0, 0)),
            pl.BlockSpec((1, HIDDEN), lambda i: (0, 0)),
        ],
        out_specs=pl.BlockSpec((_ROW_BLOCK, HIDDEN), lambda i: (i, 0)),
        out_shape=jax.ShapeDtypeStruct((VOCAB, HIDDEN), jnp.float32),
    )(embed_table, W, b.reshape(1, HIDDEN))


def _make_gather():
    info = plsc.get_sparse_core_info()
    nc, ns = info.num_cores, info.num_subcores
    nw = nc * ns
    b_per_w = N_IDS // nw
    chunk = 160
    nbuf = 4
    n_chunks = b_per_w // chunk
    n_groups = n_chunks // nbuf
    mesh = plsc.VectorSubcoreMesh(core_axis_name="c", subcore_axis_name="s")

    scratch = [pltpu.VMEM((b_per_w,), jnp.int32)]
    scratch += [pltpu.VMEM((chunk, HIDDEN), jnp.float32) for _ in range(nbuf)]
    scratch += [pltpu.SemaphoreType.DMA for _ in range(2 * nbuf)]

    @functools.partial(
        pl.kernel,
        mesh=mesh,
        out_type=jax.ShapeDtypeStruct((N_IDS, HIDDEN), jnp.float32),
        scratch_types=scratch,
    )
    def gather(table_hbm, idx_hbm, out_hbm, idx_v, *bufs_and_sems):
        bufs = bufs_and_sems[:nbuf]
        gsems = bufs_and_sems[nbuf:2 * nbuf]
        ssems = bufs_and_sems[2 * nbuf:]
        wid = lax.axis_index("s") * nc + lax.axis_index("c")
        base = wid * b_per_w
        pltpu.sync_copy(idx_hbm.at[pl.ds(base, b_per_w)], idx_v)

        def g_copy(i, k):
            return pltpu.make_async_copy(
                table_hbm.at[idx_v.at[pl.ds(i * chunk, chunk)]],
                bufs[k], gsems[k])

        def s_copy(i, k):
            return pltpu.make_async_copy(
                bufs[k], out_hbm.at[pl.ds(base + i * chunk, chunk)],
                ssems[k])

        for k in range(nbuf):
            g_copy(k, k).start()

        def outer(j, carry):
            for k in range(nbuf):
                i = j * nbuf + k
                g_copy(i, k).wait()
                s_copy(i, k).start()
                s_copy(i, k).wait()

                @pl.when(j < n_groups - 1)
                def _():
                    g_copy(i + nbuf, k).start()
            return carry

        lax.fori_loop(0, n_groups, outer, 0)

    return gather


_gather = _make_gather()


def kernel(input_ids, embed_table, W, b):
    eprime = _transform_table(embed_table, W, b)
    flat_ids = input_ids.T.reshape(-1).astype(jnp.int32)
    out_flat = _gather(eprime, flat_ids)
    return out_flat.reshape(SEQ, BATCH, HIDDEN).transpose(1, 0, 2)


# speedup vs baseline: 6.8947x; 1.0011x over previous
import functools

import jax
import jax.numpy as jnp
from jax import lax
from jax.experimental import pallas as pl
from jax.experimental.pallas import tpu as pltpu
from jax.experimental.pallas import tpu_sc as plsc

VOCAB = 100000
HIDDEN = 128
BATCH = 4096
SEQ = 50
N_IDS = BATCH * SEQ

_ROW_BLOCK = 10000


def _linear_body(e_ref, w_ref, b_ref, o_ref):
    acc = lax.dot_general(
        e_ref[...], w_ref[...],
        dimension_numbers=(((1,), (1,)), ((), ())),
        preferred_element_type=jnp.float32,
    )
    o_ref[...] = acc + b_ref[...]


def _transform_table(embed_table, W, b):
    grid = VOCAB // _ROW_BLOCK
    return pl.pallas_call(
        _linear_body,
        grid=(grid,),
        in_specs=[
            pl.BlockSpec((_ROW_BLOCK, HIDDEN), lambda i: (i, 0)),
            pl.BlockSpec((HIDDEN, HIDDEN), lambda i: (0, 0)),
            pl.BlockSpec((1, HIDDEN), lambda i: (0, 0)),
        ],
        out_specs=pl.BlockSpec((_ROW_BLOCK, HIDDEN), lambda i: (i, 0)),
        out_shape=jax.ShapeDtypeStruct((VOCAB, HIDDEN), jnp.float32),
    )(embed_table, W, b.reshape(1, HIDDEN))


def _make_gather():
    info = plsc.get_sparse_core_info()
    nc, ns = info.num_cores, info.num_subcores
    nw = nc * ns
    b_per_w = N_IDS // nw
    chunk = 200
    nbuf = 4
    n_chunks = b_per_w // chunk
    n_groups = n_chunks // nbuf
    mesh = plsc.VectorSubcoreMesh(core_axis_name="c", subcore_axis_name="s")

    scratch = [pltpu.VMEM((b_per_w,), jnp.int32)]
    scratch += [pltpu.VMEM((chunk, HIDDEN), jnp.float32) for _ in range(nbuf)]
    scratch += [pltpu.SemaphoreType.DMA for _ in range(2 * nbuf)]

    @functools.partial(
        pl.kernel,
        mesh=mesh,
        out_type=jax.ShapeDtypeStruct((N_IDS, HIDDEN), jnp.float32),
        scratch_types=scratch,
    )
    def gather(table_hbm, idx_hbm, out_hbm, idx_v, *bufs_and_sems):
        bufs = bufs_and_sems[:nbuf]
        gsems = bufs_and_sems[nbuf:2 * nbuf]
        ssems = bufs_and_sems[2 * nbuf:]
        wid = lax.axis_index("s") * nc + lax.axis_index("c")
        base = wid * b_per_w
        pltpu.sync_copy(idx_hbm.at[pl.ds(base, b_per_w)], idx_v)

        def g_copy(i, k):
            return pltpu.make_async_copy(
                table_hbm.at[idx_v.at[pl.ds(i * chunk, chunk)]],
                bufs[k], gsems[k])

        def s_copy(i, k):
            return pltpu.make_async_copy(
                bufs[k], out_hbm.at[pl.ds(base + i * chunk, chunk)],
                ssems[k])

        for k in range(nbuf):
            g_copy(k, k).start()

        def outer(j, carry):
            for k in range(nbuf):
                i = j * nbuf + k
                g_copy(i, k).wait()
                s_copy(i, k).start()
                s_copy(i, k).wait()

                @pl.when(j < n_groups - 1)
                def _():
                    g_copy(i + nbuf, k).start()
            return carry

        lax.fori_loop(0, n_groups, outer, 0)

    return gather


_gather = _make_gather()


def kernel(input_ids, embed_table, W, b):
    eprime = _transform_table(embed_table, W, b)
    flat_ids = input_ids.T.reshape(-1).astype(jnp.int32)
    out_flat = _gather(eprime, flat_ids)
    return out_flat.reshape(SEQ, BATCH, HIDDEN).transpose(1, 0, 2)


# speedup vs baseline: 6.9043x; 1.0014x over previous
import functools

import jax
import jax.numpy as jnp
from jax import lax
from jax.experimental import pallas as pl
from jax.experimental.pallas import tpu as pltpu
from jax.experimental.pallas import tpu_sc as plsc

VOCAB = 100000
HIDDEN = 128
BATCH = 4096
SEQ = 50
N_IDS = BATCH * SEQ

_ROW_BLOCK = 10000


def _linear_body(e_ref, w_ref, b_ref, o_ref):
    acc = lax.dot_general(
        e_ref[...], w_ref[...],
        dimension_numbers=(((1,), (1,)), ((), ())),
        preferred_element_type=jnp.float32,
    )
    o_ref[...] = acc + b_ref[...]


def _transform_table(embed_table, W, b):
    grid = VOCAB // _ROW_BLOCK
    return pl.pallas_call(
        _linear_body,
        grid=(grid,),
        in_specs=[
            pl.BlockSpec((_ROW_BLOCK, HIDDEN), lambda i: (i, 0)),
            pl.BlockSpec((HIDDEN, HIDDEN), lambda i: (0, 0)),
            pl.BlockSpec((1, HIDDEN), lambda i: (0, 0)),
        ],
        out_specs=pl.BlockSpec((_ROW_BLOCK, HIDDEN), lambda i: (i, 0)),
        out_shape=jax.ShapeDtypeStruct((VOCAB, HIDDEN), jnp.float32),
    )(embed_table, W, b.reshape(1, HIDDEN))


def _make_gather():
    info = plsc.get_sparse_core_info()
    nc, ns = info.num_cores, info.num_subcores
    nw = nc * ns
    b_per_w = N_IDS // nw
    chunk = 128
    nbuf = 5
    n_chunks = b_per_w // chunk
    n_groups = n_chunks // nbuf
    mesh = plsc.VectorSubcoreMesh(core_axis_name="c", subcore_axis_name="s")

    scratch = [pltpu.VMEM((b_per_w,), jnp.int32)]
    scratch += [pltpu.VMEM((chunk, HIDDEN), jnp.float32) for _ in range(nbuf)]
    scratch += [pltpu.SemaphoreType.DMA for _ in range(2 * nbuf)]

    @functools.partial(
        pl.kernel,
        mesh=mesh,
        out_type=jax.ShapeDtypeStruct((N_IDS, HIDDEN), jnp.float32),
        scratch_types=scratch,
    )
    def gather(table_hbm, idx_hbm, out_hbm, idx_v, *bufs_and_sems):
        bufs = bufs_and_sems[:nbuf]
        gsems = bufs_and_sems[nbuf:2 * nbuf]
        ssems = bufs_and_sems[2 * nbuf:]
        wid = lax.axis_index("s") * nc + lax.axis_index("c")
        base = wid * b_per_w
        pltpu.sync_copy(idx_hbm.at[pl.ds(base, b_per_w)], idx_v)

        def g_copy(i, k):
            return pltpu.make_async_copy(
                table_hbm.at[idx_v.at[pl.ds(i * chunk, chunk)]],
                bufs[k], gsems[k])

        def s_copy(i, k):
            return pltpu.make_async_copy(
                bufs[k], out_hbm.at[pl.ds(base + i * chunk, chunk)],
                ssems[k])

        for k in range(nbuf):
            g_copy(k, k).start()

        def outer(j, carry):
            for k in range(nbuf):
                i = j * nbuf + k
                g_copy(i, k).wait()
                s_copy(i, k).start()
                s_copy(i, k).wait()

                @pl.when(j < n_groups - 1)
                def _():
                    g_copy(i + nbuf, k).start()
            return carry

        lax.fori_loop(0, n_groups, outer, 0)

    return gather


_gather = _make_gather()


def kernel(input_ids, embed_table, W, b):
    eprime = _transform_table(embed_table, W, b)
    flat_ids = input_ids.T.reshape(-1).astype(jnp.int32)
    out_flat = _gather(eprime, flat_ids)
    return out_flat.reshape(SEQ, BATCH, HIDDEN).transpose(1, 0, 2)


# speedup vs baseline: 6.9758x; 1.0104x over previous
import functools

import jax
import jax.numpy as jnp
from jax import lax
from jax.experimental import pallas as pl
from jax.experimental.pallas import tpu as pltpu
from jax.experimental.pallas import tpu_sc as plsc

VOCAB = 100000
HIDDEN = 128
BATCH = 4096
SEQ = 50
N_IDS = BATCH * SEQ

_ROW_BLOCK = 20000


def _linear_body(e_ref, w_ref, b_ref, o_ref):
    acc = lax.dot_general(
        e_ref[...], w_ref[...],
        dimension_numbers=(((1,), (1,)), ((), ())),
        preferred_element_type=jnp.float32,
    )
    o_ref[...] = acc + b_ref[...]


def _transform_table(embed_table, W, b):
    grid = VOCAB // _ROW_BLOCK
    return pl.pallas_call(
        _linear_body,
        grid=(grid,),
        in_specs=[
            pl.BlockSpec((_ROW_BLOCK, HIDDEN), lambda i: (i, 0)),
            pl.BlockSpec((HIDDEN, HIDDEN), lambda i: (0, 0)),
            pl.BlockSpec((1, HIDDEN), lambda i: (0, 0)),
        ],
        out_specs=pl.BlockSpec((_ROW_BLOCK, HIDDEN), lambda i: (i, 0)),
        out_shape=jax.ShapeDtypeStruct((VOCAB, HIDDEN), jnp.float32),
    )(embed_table, W, b.reshape(1, HIDDEN))


def _make_gather():
    info = plsc.get_sparse_core_info()
    nc, ns = info.num_cores, info.num_subcores
    nw = nc * ns
    b_per_w = N_IDS // nw
    chunk = 128
    nbuf = 5
    n_chunks = b_per_w // chunk
    n_groups = n_chunks // nbuf
    mesh = plsc.VectorSubcoreMesh(core_axis_name="c", subcore_axis_name="s")

    scratch = [pltpu.VMEM((b_per_w,), jnp.int32)]
    scratch += [pltpu.VMEM((chunk, HIDDEN), jnp.float32) for _ in range(nbuf)]
    scratch += [pltpu.SemaphoreType.DMA for _ in range(2 * nbuf)]

    @functools.partial(
        pl.kernel,
        mesh=mesh,
        out_type=jax.ShapeDtypeStruct((N_IDS, HIDDEN), jnp.float32),
        scratch_types=scratch,
    )
    def gather(table_hbm, idx_hbm, out_hbm, idx_v, *bufs_and_sems):
        bufs = bufs_and_sems[:nbuf]
        gsems = bufs_and_sems[nbuf:2 * nbuf]
        ssems = bufs_and_sems[2 * nbuf:]
        wid = lax.axis_index("s") * nc + lax.axis_index("c")
        base = wid * b_per_w
        pltpu.sync_copy(idx_hbm.at[pl.ds(base, b_per_w)], idx_v)

        def g_copy(i, k):
            return pltpu.make_async_copy(
                table_hbm.at[idx_v.at[pl.ds(i * chunk, chunk)]],
                bufs[k], gsems[k])

        def s_copy(i, k):
            return pltpu.make_async_copy(
                bufs[k], out_hbm.at[pl.ds(base + i * chunk, chunk)],
                ssems[k])

        for k in range(nbuf):
            g_copy(k, k).start()

        def outer(j, carry):
            for k in range(nbuf):
                i = j * nbuf + k
                g_copy(i, k).wait()
                s_copy(i, k).start()
                s_copy(i, k).wait()

                @pl.when(j < n_groups - 1)
                def _():
                    g_copy(i + nbuf, k).start()
            return carry

        lax.fori_loop(0, n_groups, outer, 0)

    return gather


_gather = _make_gather()


def kernel(input_ids, embed_table, W, b):
    eprime = _transform_table(embed_table, W, b)
    flat_ids = input_ids.T.reshape(-1).astype(jnp.int32)
    out_flat = _gather(eprime, flat_ids)
    return out_flat.reshape(SEQ, BATCH, HIDDEN).transpose(1, 0, 2)
